# Initial kernel scaffold; baseline (speedup 1.0000x reference)
import jax
import jax.numpy as jnp
from jax.experimental import pallas as pl


def kernel(x, edge_index, edge_weight, W1, b1, W2, b2, proj1, proj2):
    raise NotImplementedError("write your pallas kernel here")


# speedup vs baseline: 12.6076x; 12.6076x over previous
import dataclasses
import functools

import jax
import jax.numpy as jnp
from jax import lax
from jax.experimental import pallas as pl
from jax.experimental.pallas import tpu as pltpu
from jax.experimental.pallas import tpu_sc as plsc

N = 10000
E = 320000
D = 128
H = 8
K = 256

NC, NS, L = 2, 16, 16
NW = NC * NS
NP = 10240
HALF = NP // NC
ROWS = 2560
EP = ROWS * 128
CH = 16
NCHUNK = ROWS // CH
HI = jax.lax.Precision.HIGHEST

_cp = pltpu.CompilerParams()
if "needs_layout_passes" in pltpu.CompilerParams.__dataclass_fields__:
    _cp = dataclasses.replace(_cp, needs_layout_passes=False)

_mesh = functools.partial(
    pl.kernel,
    mesh=plsc.VectorSubcoreMesh(core_axis_name="c", subcore_axis_name="s"),
    compiler_params=_cp,
)


def _zero_shared(shared_ref, zbuf, sid, nwords_per_tile):
    nz = zbuf.shape[0]

    @pl.loop(0, nz // L)
    def _(i):
        zbuf[pl.ds(i * L, L)] = jnp.zeros((L,), jnp.float32)

    @pl.loop(0, nwords_per_tile // nz)
    def _(i):
        pltpu.sync_copy(
            zbuf, shared_ref.at[pl.ds(sid * nwords_per_tile + i * nz, nz)])


def _sc_deg(dstM, wM):
    @_mesh(
        out_type=jax.ShapeDtypeStruct((NP,), jnp.float32),
        scratch_types=[
            pltpu.VMEM((CH, 128), jnp.int32),
            pltpu.VMEM((CH, 128), jnp.float32),
            pltpu.VMEM((640,), jnp.float32),
            pltpu.VMEM_SHARED((NP,), jnp.float32),
            pltpu.SemaphoreType.DMA,
        ],
    )
    def k(dst_hbm, w_hbm, deg_hbm, dstv, wv, zbuf, degS, sem):
        cid = lax.axis_index("c")
        sid = lax.axis_index("s")
        _zero_shared(degS, zbuf, sid, NP // NS)
        plsc.subcore_barrier()

        @pl.loop(0, NCHUNK // NS)
        def _(i):
            row0 = (sid * (NCHUNK // NS) + i) * CH
            pltpu.async_copy(dst_hbm.at[pl.ds(row0, CH)], dstv, sem).wait()
            pltpu.async_copy(w_hbm.at[pl.ds(row0, CH)], wv, sem).wait()
            for r in range(CH):
                pltpu.sync_copy(wv.at[r], degS.at[dstv.at[r]], add=True)

        plsc.subcore_barrier()
        pltpu.sync_copy(degS.at[pl.ds(cid * HALF, HALF)],
                        deg_hbm.at[pl.ds(cid * HALF, HALF)])

    return k(dstM, wM)


def _sc_norm_ht(srcM, dstM, wM, dinv, tflat):
    @_mesh(
        out_type=(
            jax.ShapeDtypeStruct((ROWS, 128), jnp.float32),
            jax.ShapeDtypeStruct((NC, H * NP), jnp.float32),
        ),
        scratch_types=[
            pltpu.VMEM((CH, 128), jnp.int32),
            pltpu.VMEM((CH, 128), jnp.int32),
            pltpu.VMEM((CH, 128), jnp.float32),
            pltpu.VMEM((CH, 128), jnp.float32),
            pltpu.VMEM((NP,), jnp.float32),
            pltpu.VMEM((NP * H,), jnp.float32),
            pltpu.VMEM((H, CH, 128), jnp.float32),
            pltpu.VMEM((640,), jnp.float32),
        ] + [pltpu.VMEM_SHARED((NP,), jnp.float32) for _ in range(H)]
        + [pltpu.SemaphoreType.DMA],
    )
    def k(src_hbm, dst_hbm, w_hbm, dinv_hbm, t_hbm, norm_hbm, ht_hbm,
          srcv, dstv, wv, normv, dinvv, tv, stage, zbuf, *rest):
        hts, sem = rest[:H], rest[H]
        cid = lax.axis_index("c")
        sid = lax.axis_index("s")
        for j in range(H):
            _zero_shared(hts[j], zbuf, sid, NP // NS)
        pltpu.async_copy(dinv_hbm, dinvv, sem).wait()
        pltpu.async_copy(t_hbm, tv, sem).wait()
        plsc.subcore_barrier()

        nch = NCHUNK // NW

        @pl.loop(0, nch)
        def _(i):
            row0 = (cid * (NCHUNK // NC) + sid * nch + i) * CH
            pltpu.async_copy(src_hbm.at[pl.ds(row0, CH)], srcv, sem).wait()
            pltpu.async_copy(dst_hbm.at[pl.ds(row0, CH)], dstv, sem).wait()
            pltpu.async_copy(w_hbm.at[pl.ds(row0, CH)], wv, sem).wait()

            @pl.loop(0, CH)
            def _(r):
                @pl.loop(0, 128 // L)
                def _(g):
                    s16 = srcv[r, pl.ds(g * L, L)]
                    d16 = dstv[r, pl.ds(g * L, L)]
                    w16 = wv[r, pl.ds(g * L, L)]
                    nrm = (plsc.load_gather(dinvv, [s16]) * w16
                           * plsc.load_gather(dinvv, [d16]))
                    normv[r, pl.ds(g * L, L)] = nrm
                    s8 = s16 * 8
                    for j in range(H):
                        tvj = plsc.load_gather(tv, [s8 + j])
                        stage[j, r, pl.ds(g * L, L)] = tvj * nrm

            pltpu.async_copy(normv, norm_hbm.at[pl.ds(row0, CH)], sem).wait()
            for j in range(H):
                for r in range(CH):
                    pltpu.sync_copy(stage.at[j, r], hts[j].at[dstv.at[r]],
                                    add=True)

        plsc.subcore_barrier()
        for j in range(H):
            pltpu.sync_copy(
                hts[j].at[pl.ds(sid * (NP // NS), NP // NS)],
                ht_hbm.at[cid, pl.ds(j * NP + sid * (NP // NS), NP // NS)])

    return k(srcM, dstM, wM, dinv, tflat)


def _sc_m1(srcM, dstM, normM, codes1, iota2):
    @_mesh(
        out_type=(
            jax.ShapeDtypeStruct((NC, K * HALF), jnp.float32),
            jax.ShapeDtypeStruct((NC, K), jnp.float32),
        ),
        scratch_types=[
            pltpu.VMEM((CH, 128), jnp.int32),
            pltpu.VMEM((CH, 128), jnp.int32),
            pltpu.VMEM((CH, 128), jnp.float32),
            pltpu.VMEM((CH, 128), jnp.int32),
            pltpu.VMEM((CH, 128), jnp.float32),
            pltpu.VMEM((NP,), jnp.int32),
            pltpu.VMEM((2, 128), jnp.int32),
            pltpu.VMEM((K,), jnp.float32),
            pltpu.VMEM((8192,), jnp.float32),
            pltpu.VMEM_SHARED((K * HALF,), jnp.float32),
            pltpu.VMEM_SHARED((K,), jnp.float32),
            pltpu.SemaphoreType.DMA,
        ],
    )
    def k(src_hbm, dst_hbm, norm_hbm, c1_hbm, iota_hbm, m1_hbm, cnt_hbm,
          srcv, dstv, normv, flatv, valv, c1v, iotav, cntv, zbuf,
          m1S, cntS, sem):
        cid = lax.axis_index("c")
        sid = lax.axis_index("s")
        _zero_shared(m1S, zbuf, sid, K * HALF // NS)

        @pl.when(sid == 0)
        def _():
            @pl.loop(0, K // L)
            def _(i):
                zbuf[pl.ds(i * L, L)] = jnp.zeros((L,), jnp.float32)
            pltpu.sync_copy(zbuf.at[pl.ds(0, K)], cntS)

        pltpu.async_copy(c1_hbm, c1v, sem).wait()
        pltpu.async_copy(iota_hbm, iotav, sem).wait()
        plsc.subcore_barrier()
        base = cid * HALF

        @pl.loop(0, K // L)
        def _(i):
            cntv[pl.ds(i * L, L)] = jnp.zeros((L,), jnp.float32)

        wid = cid * NS + sid

        @pl.loop(0, (NP // NW) // L)
        def _(i):
            n0 = wid * (NP // NW) + i * L
            c16 = c1v[pl.ds(n0, L)]
            valid = (lax.iota(jnp.int32, L) + n0) < N
            plsc.addupdate_scatter(
                cntv, [c16], jnp.where(valid, 1.0, 0.0))

        @pl.loop(0, NCHUNK // NS)
        def _(i):
            row0 = (sid * (NCHUNK // NS) + i) * CH
            pltpu.async_copy(src_hbm.at[pl.ds(row0, CH)], srcv, sem).wait()
            pltpu.async_copy(dst_hbm.at[pl.ds(row0, CH)], dstv, sem).wait()
            pltpu.async_copy(norm_hbm.at[pl.ds(row0, CH)], normv, sem).wait()

            @pl.loop(0, CH)
            def _(r):
                @pl.loop(0, 128 // L)
                def _(g):
                    s16 = srcv[r, pl.ds(g * L, L)]
                    d16 = dstv[r, pl.ds(g * L, L)]
                    n16 = normv[r, pl.ds(g * L, L)]
                    c16 = plsc.load_gather(c1v, [d16])
                    col = s16 - base
                    owned = (col >= 0) & (col < HALF)
                    col = jnp.clip(col, 0, HALF - 1)
                    flatv[r, pl.ds(g * L, L)] = c16 * HALF + col
                    valv[r, pl.ds(g * L, L)] = jnp.where(owned, n16, 0.0)

            for r in range(CH):
                pltpu.sync_copy(valv.at[r], m1S.at[flatv.at[r]], add=True)

        for r in range(2):
            pltpu.sync_copy(cntv.at[pl.ds(r * 128, 128)],
                            cntS.at[iotav.at[r]], add=True)

        plsc.subcore_barrier()
        W = K * HALF // NS
        pltpu.sync_copy(m1S.at[pl.ds(sid * W, W)],
                        m1_hbm.at[cid, pl.ds(sid * W, W)])

        @pl.when(sid == 0)
        def _():
            pltpu.sync_copy(cntS, cnt_hbm.at[cid])

    return k(srcM, dstM, normM, codes1, iota2)


def _sc_gt(srcM, dstM, normM, codes1, uflat):
    @_mesh(
        out_type=jax.ShapeDtypeStruct((NC, H * NP), jnp.float32),
        scratch_types=[
            pltpu.VMEM((CH, 128), jnp.int32),
            pltpu.VMEM((CH, 128), jnp.int32),
            pltpu.VMEM((CH, 128), jnp.float32),
            pltpu.VMEM((NP,), jnp.int32),
            pltpu.VMEM((K * H,), jnp.float32),
            pltpu.VMEM((H, CH, 128), jnp.float32),
            pltpu.VMEM((640,), jnp.float32),
        ] + [pltpu.VMEM_SHARED((NP,), jnp.float32) for _ in range(H)]
        + [pltpu.SemaphoreType.DMA],
    )
    def k(src_hbm, dst_hbm, norm_hbm, c1_hbm, u_hbm, gt_hbm,
          srcv, dstv, normv, c1v, uv, stage, zbuf, *rest):
        gts, sem = rest[:H], rest[H]
        cid = lax.axis_index("c")
        sid = lax.axis_index("s")
        for j in range(H):
            _zero_shared(gts[j], zbuf, sid, NP // NS)
        pltpu.async_copy(c1_hbm, c1v, sem).wait()
        pltpu.async_copy(u_hbm, uv, sem).wait()
        plsc.subcore_barrier()

        nch = NCHUNK // NW

        @pl.loop(0, nch)
        def _(i):
            row0 = (cid * (NCHUNK // NC) + sid * nch + i) * CH
            pltpu.async_copy(src_hbm.at[pl.ds(row0, CH)], srcv, sem).wait()
            pltpu.async_copy(dst_hbm.at[pl.ds(row0, CH)], dstv, sem).wait()
            pltpu.async_copy(norm_hbm.at[pl.ds(row0, CH)], normv, sem).wait()

            @pl.loop(0, CH)
            def _(r):
                @pl.loop(0, 128 // L)
                def _(g):
                    s16 = srcv[r, pl.ds(g * L, L)]
                    n16 = normv[r, pl.ds(g * L, L)]
                    k16 = plsc.load_gather(c1v, [s16]) * 8
                    for j in range(H):
                        uvj = plsc.load_gather(uv, [k16 + j])
                        stage[j, r, pl.ds(g * L, L)] = uvj * n16

            for j in range(H):
                for r in range(CH):
                    pltpu.sync_copy(stage.at[j, r], gts[j].at[dstv.at[r]],
                                    add=True)

        plsc.subcore_barrier()
        for j in range(H):
            pltpu.sync_copy(
                gts[j].at[pl.ds(sid * (NP // NS), NP // NS)],
                gt_hbm.at[cid, pl.ds(j * NP + sid * (NP // NS), NP // NS)])

    return k(srcM, dstM, normM, codes1, uflat)


def _sc_b(srcM, dstM, normM, codes1, codes2, iota2):
    @_mesh(
        out_type=(
            jax.ShapeDtypeStruct((NC, K * K), jnp.float32),
            jax.ShapeDtypeStruct((NC, K), jnp.float32),
        ),
        scratch_types=[
            pltpu.VMEM((CH, 128), jnp.int32),
            pltpu.VMEM((CH, 128), jnp.int32),
            pltpu.VMEM((CH, 128), jnp.float32),
            pltpu.VMEM((CH, 128), jnp.int32),
            pltpu.VMEM((NP,), jnp.int32),
            pltpu.VMEM((NP,), jnp.int32),
            pltpu.VMEM((2, 128), jnp.int32),
            pltpu.VMEM((K,), jnp.float32),
            pltpu.VMEM((4096,), jnp.float32),
            pltpu.VMEM_SHARED((K * K,), jnp.float32),
            pltpu.VMEM_SHARED((K,), jnp.float32),
            pltpu.SemaphoreType.DMA,
        ],
    )
    def k(src_hbm, dst_hbm, norm_hbm, c1_hbm, c2_hbm, iota_hbm,
          b_hbm, cnt_hbm,
          srcv, dstv, normv, flatv, c1v, c2v, iotav, cntv, zbuf,
          bS, cntS, sem):
        cid = lax.axis_index("c")
        sid = lax.axis_index("s")
        _zero_shared(bS, zbuf, sid, K * K // NS)

        @pl.when(sid == 0)
        def _():
            pltpu.sync_copy(zbuf.at[pl.ds(0, K)], cntS)

        pltpu.async_copy(c1_hbm, c1v, sem).wait()
        pltpu.async_copy(c2_hbm, c2v, sem).wait()
        pltpu.async_copy(iota_hbm, iotav, sem).wait()
        plsc.subcore_barrier()

        @pl.loop(0, K // L)
        def _(i):
            cntv[pl.ds(i * L, L)] = jnp.zeros((L,), jnp.float32)

        wid = cid * NS + sid

        @pl.loop(0, (NP // NW) // L)
        def _(i):
            n0 = wid * (NP // NW) + i * L
            c16 = c2v[pl.ds(n0, L)]
            valid = (lax.iota(jnp.int32, L) + n0) < N
            plsc.addupdate_scatter(
                cntv, [c16], jnp.where(valid, 1.0, 0.0))

        nch = NCHUNK // NW

        @pl.loop(0, nch)
        def _(i):
            row0 = (cid * (NCHUNK // NC) + sid * nch + i) * CH
            pltpu.async_copy(src_hbm.at[pl.ds(row0, CH)], srcv, sem).wait()
            pltpu.async_copy(dst_hbm.at[pl.ds(row0, CH)], dstv, sem).wait()
            pltpu.async_copy(norm_hbm.at[pl.ds(row0, CH)], normv, sem).wait()

            @pl.loop(0, CH)
            def _(r):
                @pl.loop(0, 128 // L)
                def _(g):
                    s16 = srcv[r, pl.ds(g * L, L)]
                    d16 = dstv[r, pl.ds(g * L, L)]
                    k1 = plsc.load_gather(c1v, [s16])
                    k2 = plsc.load_gather(c2v, [d16])
                    flatv[r, pl.ds(g * L, L)] = k2 * K + k1

            for r in range(CH):
                pltpu.sync_copy(normv.at[r], bS.at[flatv.at[r]], add=True)

        for r in range(2):
            pltpu.sync_copy(cntv.at[pl.ds(r * 128, 128)],
                            cntS.at[iotav.at[r]], add=True)

        plsc.subcore_barrier()
        W = K * K // NS
        pltpu.sync_copy(bS.at[pl.ds(sid * W, W)],
                        b_hbm.at[cid, pl.ds(sid * W, W)])

        @pl.when(sid == 0)
        def _():
            pltpu.sync_copy(cntS, cnt_hbm.at[cid])

    return k(srcM, dstM, normM, codes1, codes2, iota2)


def _sc_recon(codes2, lsm):
    RPT = NP // NW
    CK = 64

    @_mesh(
        out_type=jax.ShapeDtypeStruct((NP, D), jnp.float32),
        scratch_types=[
            pltpu.VMEM((RPT,), jnp.int32),
            pltpu.VMEM((CK, D), jnp.float32),
            pltpu.SemaphoreType.DMA,
        ],
    )
    def k(c2_hbm, lsm_hbm, out_hbm, c2v, bufv, sem):
        cid = lax.axis_index("c")
        sid = lax.axis_index("s")
        wid = cid * NS + sid
        pltpu.sync_copy(c2_hbm.at[pl.ds(wid * RPT, RPT)], c2v)
        for kk in range(RPT // CK):
            pltpu.async_copy(lsm_hbm.at[c2v.at[pl.ds(kk * CK, CK)]],
                             bufv, sem).wait()
            pltpu.sync_copy(bufv,
                            out_hbm.at[pl.ds(wid * RPT + kk * CK, CK)])

    return k(codes2, lsm)


def _tc_t(xp, proj1):
    def body(x_ref, p_ref, o_ref):
        o_ref[...] = lax.dot_general(
            x_ref[...], p_ref[...], (((1,), (0,)), ((), ())),
            precision=HI, preferred_element_type=jnp.float32)

    return pl.pallas_call(
        body,
        out_shape=jax.ShapeDtypeStruct((NP, H), jnp.float32),
    )(xp, proj1)


def _tc_dinv(degM):
    def body(d_ref, o_ref):
        d = d_ref[...]
        o_ref[...] = jnp.where(d > 0, lax.rsqrt(jnp.maximum(d, 1e-12)), 0.0)

    return pl.pallas_call(
        body,
        out_shape=jax.ShapeDtypeStruct(degM.shape, jnp.float32),
    )(degM)


def _tc_codes(htp3):
    def body(h_ref, o_ref):
        a = h_ref[...]
        c = jnp.zeros((NP // 128, 128), jnp.int32)
        for j in range(H):
            bit = ((a[j] + a[H + j]) > 0).astype(jnp.int32)
            c = c + (bit << j)
        o_ref[...] = c

    return pl.pallas_call(
        body,
        out_shape=jax.ShapeDtypeStruct((NP // 128, 128), jnp.int32),
    )(htp3)


def _tc_stage1(M1h, xpad2, counts1p, W1, b1, proj2):
    NB = 10
    ngrid = NC * NB

    def body(m_ref, x_ref, cnt_ref, w1_ref, b1_ref, p2_ref,
             hc_ref, u_ref, acc_ref):
        i = pl.program_id(0)

        @pl.when(i == 0)
        def _():
            acc_ref[...] = jnp.zeros((K, D), jnp.float32)

        acc_ref[...] += lax.dot_general(
            m_ref[0], x_ref[0], (((1,), (0,)), ((), ())),
            precision=HI, preferred_element_type=jnp.float32)

        @pl.when(i == ngrid - 1)
        def _():
            cnt = cnt_ref[0] + cnt_ref[1]
            means = acc_ref[...] / jnp.clip(cnt, 1.0)[:, None]
            hc = lax.dot_general(
                means, w1_ref[...], (((1,), (1,)), ((), ())),
                precision=HI, preferred_element_type=jnp.float32)
            hc = jnp.maximum(hc + b1_ref[...][None, :], 0.0)
            hc_ref[...] = hc
            u_ref[...] = lax.dot_general(
                hc, p2_ref[...], (((1,), (0,)), ((), ())),
                precision=HI, preferred_element_type=jnp.float32)

    blk = HALF // NB
    return pl.pallas_call(
        body,
        grid=(ngrid,),
        in_specs=[
            pl.BlockSpec((1, K, blk), lambda i: (i // NB, 0, i % NB)),
            pl.BlockSpec((1, blk, D), lambda i: (i // NB, i % NB, 0)),
            pl.BlockSpec((NC, K), lambda i: (0, 0)),
            pl.BlockSpec((D, D), lambda i: (0, 0)),
            pl.BlockSpec((D,), lambda i: (0,)),
            pl.BlockSpec((D, H), lambda i: (0, 0)),
        ],
        out_specs=[
            pl.BlockSpec((K, D), lambda i: (0, 0)),
            pl.BlockSpec((K, H), lambda i: (0, 0)),
        ],
        out_shape=[
            jax.ShapeDtypeStruct((K, D), jnp.float32),
            jax.ShapeDtypeStruct((K, H), jnp.float32),
        ],
        scratch_shapes=[pltpu.VMEM((K, D), jnp.float32)],
    )(M1h, xpad2, counts1p, W1, b1, proj2)


def _tc_final(Bp, counts2p, hc, W2, b2):
    def body(b_ref, cnt_ref, hc_ref, w2_ref, b2_ref, o_ref):
        Bs = b_ref[0] + b_ref[1]
        sums2 = lax.dot_general(
            Bs, hc_ref[...], (((1,), (0,)), ((), ())),
            precision=HI, preferred_element_type=jnp.float32)
        cnt = cnt_ref[0] + cnt_ref[1]
        m2 = sums2 / jnp.clip(cnt, 1.0)[:, None]
        h2 = lax.dot_general(
            m2, w2_ref[...], (((1,), (1,)), ((), ())),
            precision=HI, preferred_element_type=jnp.float32)
        h2 = h2 + b2_ref[...][None, :]
        mx = jnp.max(h2, axis=1, keepdims=True)
        lse = jnp.log(jnp.sum(jnp.exp(h2 - mx), axis=1, keepdims=True)) + mx
        o_ref[...] = h2 - lse

    return pl.pallas_call(
        body,
        out_shape=jax.ShapeDtypeStruct((K, D), jnp.float32),
    )(Bp, counts2p, hc, W2, b2)


def kernel(x, edge_index, edge_weight, W1, b1, W2, b2, proj1, proj2):
    src = edge_index[0]
    dst = edge_index[1]
    pad = EP - E
    srcM = jnp.concatenate([src, jnp.zeros((pad,), jnp.int32)]).reshape(ROWS, 128)
    dstM = jnp.concatenate([dst, jnp.zeros((pad,), jnp.int32)]).reshape(ROWS, 128)
    wM = jnp.concatenate(
        [edge_weight, jnp.zeros((pad,), jnp.float32)]).reshape(ROWS, 128)
    xp = jnp.concatenate([x, jnp.zeros((NP - N, D), jnp.float32)])
    iota2 = jnp.arange(K, dtype=jnp.int32).reshape(2, 128)

    degp = _sc_deg(dstM, wM)
    dinv = _tc_dinv(degp.reshape(NP // 128, 128)).reshape(NP)
    t = _tc_t(xp, proj1)
    tflat = t.reshape(NP * H)

    normM, htp = _sc_norm_ht(srcM, dstM, wM, dinv, tflat)
    codes1 = _tc_codes(htp.reshape(NC * H, NP // 128, 128)).reshape(NP)

    m1flat, counts1p = _sc_m1(srcM, dstM, normM, codes1, iota2)
    hc, u = _tc_stage1(m1flat.reshape(NC, K, HALF), xp.reshape(NC, HALF, D),
                       counts1p, W1, b1, proj2)

    gtp = _sc_gt(srcM, dstM, normM, codes1, u.reshape(K * H))
    codes2 = _tc_codes(gtp.reshape(NC * H, NP // 128, 128)).reshape(NP)

    bflat, counts2p = _sc_b(srcM, dstM, normM, codes1, codes2, iota2)
    lsm = _tc_final(bflat.reshape(NC, K, K), counts2p, hc, W2, b2)

    outp = _sc_recon(codes2, lsm)
    return outp[:N]


# speedup vs baseline: 12.9583x; 1.0278x over previous
import dataclasses
import functools

import jax
import jax.numpy as jnp
from jax import lax
from jax.experimental import pallas as pl
from jax.experimental.pallas import tpu as pltpu
from jax.experimental.pallas import tpu_sc as plsc

N = 10000
E = 320000
D = 128
H = 8
K = 256

NC, NS, L = 2, 16, 16
NW = NC * NS
NP = 10240
HALF = NP // NC
ROWS = 2560
EP = ROWS * 128
CH = 16
NCHUNK = ROWS // CH
HI = jax.lax.Precision.HIGHEST

_cp = pltpu.CompilerParams()
if "needs_layout_passes" in pltpu.CompilerParams.__dataclass_fields__:
    _cp = dataclasses.replace(_cp, needs_layout_passes=False)

_mesh = functools.partial(
    pl.kernel,
    mesh=plsc.VectorSubcoreMesh(core_axis_name="c", subcore_axis_name="s"),
    compiler_params=_cp,
)


def _zero_shared(shared_ref, zbuf, sid, nwords_per_tile):
    nz = zbuf.shape[0]

    @pl.loop(0, nz // L)
    def _(i):
        zbuf[pl.ds(i * L, L)] = jnp.zeros((L,), jnp.float32)

    @pl.loop(0, nwords_per_tile // nz)
    def _(i):
        pltpu.sync_copy(
            zbuf, shared_ref.at[pl.ds(sid * nwords_per_tile + i * nz, nz)])


def _sc_deg(dstM, wM):
    @_mesh(
        out_type=jax.ShapeDtypeStruct((NP,), jnp.float32),
        scratch_types=[
            pltpu.VMEM((CH, 128), jnp.int32),
            pltpu.VMEM((CH, 128), jnp.float32),
            pltpu.VMEM((640,), jnp.float32),
            pltpu.VMEM_SHARED((NP,), jnp.float32),
            pltpu.SemaphoreType.DMA,
            pltpu.SemaphoreType.DMA,
        ],
    )
    def k(dst_hbm, w_hbm, deg_hbm, dstv, wv, zbuf, degS, sem, sem2):
        cid = lax.axis_index("c")
        sid = lax.axis_index("s")
        _zero_shared(degS, zbuf, sid, NP // NS)
        plsc.subcore_barrier()

        @pl.loop(0, NCHUNK // NS)
        def _(i):
            row0 = (sid * (NCHUNK // NS) + i) * CH
            pltpu.async_copy(dst_hbm.at[pl.ds(row0, CH)], dstv, sem).wait()
            pltpu.async_copy(w_hbm.at[pl.ds(row0, CH)], wv, sem).wait()
            ds_ = [pltpu.async_copy(wv.at[r], degS.at[dstv.at[r]], sem2,
                                    add=True) for r in range(CH)]
            for d in ds_:
                d.wait()

        plsc.subcore_barrier()
        pltpu.sync_copy(degS.at[pl.ds(cid * HALF, HALF)],
                        deg_hbm.at[pl.ds(cid * HALF, HALF)])

    return k(dstM, wM)


def _sc_norm_ht(srcM, dstM, wM, dinv, tflat):
    @_mesh(
        out_type=(
            jax.ShapeDtypeStruct((ROWS, 128), jnp.float32),
            jax.ShapeDtypeStruct((NC, H * NP), jnp.float32),
        ),
        scratch_types=[
            pltpu.VMEM((CH, 128), jnp.int32),
            pltpu.VMEM((CH, 128), jnp.int32),
            pltpu.VMEM((CH, 128), jnp.float32),
            pltpu.VMEM((CH, 128), jnp.float32),
            pltpu.VMEM((NP,), jnp.float32),
            pltpu.VMEM((NP * H,), jnp.float32),
            pltpu.VMEM((H, CH, 128), jnp.float32),
            pltpu.VMEM((640,), jnp.float32),
        ] + [pltpu.VMEM_SHARED((NP,), jnp.float32) for _ in range(H)]
        + [pltpu.SemaphoreType.DMA],
    )
    def k(src_hbm, dst_hbm, w_hbm, dinv_hbm, t_hbm, norm_hbm, ht_hbm,
          srcv, dstv, wv, normv, dinvv, tv, stage, zbuf, *rest):
        hts, sem = rest[:H], rest[H]
        cid = lax.axis_index("c")
        sid = lax.axis_index("s")
        for j in range(H):
            _zero_shared(hts[j], zbuf, sid, NP // NS)
        pltpu.async_copy(dinv_hbm, dinvv, sem).wait()
        pltpu.async_copy(t_hbm, tv, sem).wait()
        plsc.subcore_barrier()

        nch = NCHUNK // NW

        @pl.loop(0, nch)
        def _(i):
            row0 = (cid * (NCHUNK // NC) + sid * nch + i) * CH
            pltpu.async_copy(src_hbm.at[pl.ds(row0, CH)], srcv, sem).wait()
            pltpu.async_copy(dst_hbm.at[pl.ds(row0, CH)], dstv, sem).wait()
            pltpu.async_copy(w_hbm.at[pl.ds(row0, CH)], wv, sem).wait()

            @pl.loop(0, CH)
            def _(r):
                @pl.loop(0, 128 // L)
                def _(g):
                    s16 = srcv[r, pl.ds(g * L, L)]
                    d16 = dstv[r, pl.ds(g * L, L)]
                    w16 = wv[r, pl.ds(g * L, L)]
                    nrm = (plsc.load_gather(dinvv, [s16]) * w16
                           * plsc.load_gather(dinvv, [d16]))
                    normv[r, pl.ds(g * L, L)] = nrm
                    s8 = s16 * 8
                    for j in range(H):
                        tvj = plsc.load_gather(tv, [s8 + j])
                        stage[j, r, pl.ds(g * L, L)] = tvj * nrm

            pltpu.async_copy(normv, norm_hbm.at[pl.ds(row0, CH)], sem).wait()

            @pl.loop(0, CH)
            def _(r):
                ds_ = [pltpu.async_copy(stage.at[j, r],
                                        hts[j].at[dstv.at[r]],
                                        sem, add=True) for j in range(H)]
                for d in ds_:
                    d.wait()

        plsc.subcore_barrier()
        for j in range(H):
            pltpu.sync_copy(
                hts[j].at[pl.ds(sid * (NP // NS), NP // NS)],
                ht_hbm.at[cid, pl.ds(j * NP + sid * (NP // NS), NP // NS)])

    return k(srcM, dstM, wM, dinv, tflat)


def _sc_m1(srcM, dstM, normM, codes1, iota2):
    @_mesh(
        out_type=(
            jax.ShapeDtypeStruct((NC, K * HALF), jnp.float32),
            jax.ShapeDtypeStruct((NC, K), jnp.float32),
        ),
        scratch_types=[
            pltpu.VMEM((CH, 128), jnp.int32),
            pltpu.VMEM((CH, 128), jnp.int32),
            pltpu.VMEM((CH, 128), jnp.float32),
            pltpu.VMEM((CH, 128), jnp.int32),
            pltpu.VMEM((CH, 128), jnp.float32),
            pltpu.VMEM((NP,), jnp.int32),
            pltpu.VMEM((2, 128), jnp.int32),
            pltpu.VMEM((K,), jnp.float32),
            pltpu.VMEM((8192,), jnp.float32),
            pltpu.VMEM_SHARED((K * HALF,), jnp.float32),
            pltpu.VMEM_SHARED((K,), jnp.float32),
            pltpu.SemaphoreType.DMA,
        ],
    )
    def k(src_hbm, dst_hbm, norm_hbm, c1_hbm, iota_hbm, m1_hbm, cnt_hbm,
          srcv, dstv, normv, flatv, valv, c1v, iotav, cntv, zbuf,
          m1S, cntS, sem):
        cid = lax.axis_index("c")
        sid = lax.axis_index("s")
        _zero_shared(m1S, zbuf, sid, K * HALF // NS)

        @pl.when(sid == 0)
        def _():
            @pl.loop(0, K // L)
            def _(i):
                zbuf[pl.ds(i * L, L)] = jnp.zeros((L,), jnp.float32)
            pltpu.sync_copy(zbuf.at[pl.ds(0, K)], cntS)

        pltpu.async_copy(c1_hbm, c1v, sem).wait()
        pltpu.async_copy(iota_hbm, iotav, sem).wait()
        plsc.subcore_barrier()
        base = cid * HALF

        @pl.loop(0, K // L)
        def _(i):
            cntv[pl.ds(i * L, L)] = jnp.zeros((L,), jnp.float32)

        wid = cid * NS + sid

        @pl.loop(0, (NP // NW) // L)
        def _(i):
            n0 = wid * (NP // NW) + i * L
            c16 = c1v[pl.ds(n0, L)]
            valid = (lax.iota(jnp.int32, L) + n0) < N
            plsc.addupdate_scatter(
                cntv, [c16], jnp.where(valid, 1.0, 0.0))

        @pl.loop(0, NCHUNK // NS)
        def _(i):
            row0 = (sid * (NCHUNK // NS) + i) * CH
            pltpu.async_copy(src_hbm.at[pl.ds(row0, CH)], srcv, sem).wait()
            pltpu.async_copy(dst_hbm.at[pl.ds(row0, CH)], dstv, sem).wait()
            pltpu.async_copy(norm_hbm.at[pl.ds(row0, CH)], normv, sem).wait()

            @pl.loop(0, CH)
            def _(r):
                @pl.loop(0, 128 // L)
                def _(g):
                    s16 = srcv[r, pl.ds(g * L, L)]
                    d16 = dstv[r, pl.ds(g * L, L)]
                    n16 = normv[r, pl.ds(g * L, L)]
                    c16 = plsc.load_gather(c1v, [d16])
                    col = s16 - base
                    owned = (col >= 0) & (col < HALF)
                    col = jnp.clip(col, 0, HALF - 1)
                    flatv[r, pl.ds(g * L, L)] = c16 * HALF + col
                    valv[r, pl.ds(g * L, L)] = jnp.where(owned, n16, 0.0)

            ds_ = [pltpu.async_copy(valv.at[r], m1S.at[flatv.at[r]], sem,
                                    add=True) for r in range(CH)]
            for d in ds_:
                d.wait()

        for r in range(2):
            pltpu.sync_copy(cntv.at[pl.ds(r * 128, 128)],
                            cntS.at[iotav.at[r]], add=True)

        plsc.subcore_barrier()
        W = K * HALF // NS
        pltpu.sync_copy(m1S.at[pl.ds(sid * W, W)],
                        m1_hbm.at[cid, pl.ds(sid * W, W)])

        @pl.when(sid == 0)
        def _():
            pltpu.sync_copy(cntS, cnt_hbm.at[cid])

    return k(srcM, dstM, normM, codes1, iota2)


def _sc_gt(srcM, dstM, normM, codes1, uflat):
    @_mesh(
        out_type=jax.ShapeDtypeStruct((NC, H * NP), jnp.float32),
        scratch_types=[
            pltpu.VMEM((CH, 128), jnp.int32),
            pltpu.VMEM((CH, 128), jnp.int32),
            pltpu.VMEM((CH, 128), jnp.float32),
            pltpu.VMEM((NP,), jnp.int32),
            pltpu.VMEM((K * H,), jnp.float32),
            pltpu.VMEM((H, CH, 128), jnp.float32),
            pltpu.VMEM((640,), jnp.float32),
        ] + [pltpu.VMEM_SHARED((NP,), jnp.float32) for _ in range(H)]
        + [pltpu.SemaphoreType.DMA],
    )
    def k(src_hbm, dst_hbm, norm_hbm, c1_hbm, u_hbm, gt_hbm,
          srcv, dstv, normv, c1v, uv, stage, zbuf, *rest):
        gts, sem = rest[:H], rest[H]
        cid = lax.axis_index("c")
        sid = lax.axis_index("s")
        for j in range(H):
            _zero_shared(gts[j], zbuf, sid, NP // NS)
        pltpu.async_copy(c1_hbm, c1v, sem).wait()
        pltpu.async_copy(u_hbm, uv, sem).wait()
        plsc.subcore_barrier()

        nch = NCHUNK // NW

        @pl.loop(0, nch)
        def _(i):
            row0 = (cid * (NCHUNK // NC) + sid * nch + i) * CH
            pltpu.async_copy(src_hbm.at[pl.ds(row0, CH)], srcv, sem).wait()
            pltpu.async_copy(dst_hbm.at[pl.ds(row0, CH)], dstv, sem).wait()
            pltpu.async_copy(norm_hbm.at[pl.ds(row0, CH)], normv, sem).wait()

            @pl.loop(0, CH)
            def _(r):
                @pl.loop(0, 128 // L)
                def _(g):
                    s16 = srcv[r, pl.ds(g * L, L)]
                    n16 = normv[r, pl.ds(g * L, L)]
                    k16 = plsc.load_gather(c1v, [s16]) * 8
                    for j in range(H):
                        uvj = plsc.load_gather(uv, [k16 + j])
                        stage[j, r, pl.ds(g * L, L)] = uvj * n16

            @pl.loop(0, CH)
            def _(r):
                ds_ = [pltpu.async_copy(stage.at[j, r],
                                        gts[j].at[dstv.at[r]],
                                        sem, add=True) for j in range(H)]
                for d in ds_:
                    d.wait()

        plsc.subcore_barrier()
        for j in range(H):
            pltpu.sync_copy(
                gts[j].at[pl.ds(sid * (NP // NS), NP // NS)],
                gt_hbm.at[cid, pl.ds(j * NP + sid * (NP // NS), NP // NS)])

    return k(srcM, dstM, normM, codes1, uflat)


def _sc_b(srcM, dstM, normM, codes1, codes2, iota2):
    @_mesh(
        out_type=(
            jax.ShapeDtypeStruct((NC, K * K), jnp.float32),
            jax.ShapeDtypeStruct((NC, K), jnp.float32),
        ),
        scratch_types=[
            pltpu.VMEM((CH, 128), jnp.int32),
            pltpu.VMEM((CH, 128), jnp.int32),
            pltpu.VMEM((CH, 128), jnp.float32),
            pltpu.VMEM((CH, 128), jnp.int32),
            pltpu.VMEM((NP,), jnp.int32),
            pltpu.VMEM((NP,), jnp.int32),
            pltpu.VMEM((2, 128), jnp.int32),
            pltpu.VMEM((K,), jnp.float32),
            pltpu.VMEM((4096,), jnp.float32),
            pltpu.VMEM_SHARED((K * K,), jnp.float32),
            pltpu.VMEM_SHARED((K,), jnp.float32),
            pltpu.SemaphoreType.DMA,
        ],
    )
    def k(src_hbm, dst_hbm, norm_hbm, c1_hbm, c2_hbm, iota_hbm,
          b_hbm, cnt_hbm,
          srcv, dstv, normv, flatv, c1v, c2v, iotav, cntv, zbuf,
          bS, cntS, sem):
        cid = lax.axis_index("c")
        sid = lax.axis_index("s")
        _zero_shared(bS, zbuf, sid, K * K // NS)

        @pl.when(sid == 0)
        def _():
            pltpu.sync_copy(zbuf.at[pl.ds(0, K)], cntS)

        pltpu.async_copy(c1_hbm, c1v, sem).wait()
        pltpu.async_copy(c2_hbm, c2v, sem).wait()
        pltpu.async_copy(iota_hbm, iotav, sem).wait()
        plsc.subcore_barrier()

        @pl.loop(0, K // L)
        def _(i):
            cntv[pl.ds(i * L, L)] = jnp.zeros((L,), jnp.float32)

        wid = cid * NS + sid

        @pl.loop(0, (NP // NW) // L)
        def _(i):
            n0 = wid * (NP // NW) + i * L
            c16 = c2v[pl.ds(n0, L)]
            valid = (lax.iota(jnp.int32, L) + n0) < N
            plsc.addupdate_scatter(
                cntv, [c16], jnp.where(valid, 1.0, 0.0))

        nch = NCHUNK // NW

        @pl.loop(0, nch)
        def _(i):
            row0 = (cid * (NCHUNK // NC) + sid * nch + i) * CH
            pltpu.async_copy(src_hbm.at[pl.ds(row0, CH)], srcv, sem).wait()
            pltpu.async_copy(dst_hbm.at[pl.ds(row0, CH)], dstv, sem).wait()
            pltpu.async_copy(norm_hbm.at[pl.ds(row0, CH)], normv, sem).wait()

            @pl.loop(0, CH)
            def _(r):
                @pl.loop(0, 128 // L)
                def _(g):
                    s16 = srcv[r, pl.ds(g * L, L)]
                    d16 = dstv[r, pl.ds(g * L, L)]
                    k1 = plsc.load_gather(c1v, [s16])
                    k2 = plsc.load_gather(c2v, [d16])
                    flatv[r, pl.ds(g * L, L)] = k2 * K + k1

            ds_ = [pltpu.async_copy(normv.at[r], bS.at[flatv.at[r]], sem,
                                    add=True) for r in range(CH)]
            for d in ds_:
                d.wait()

        for r in range(2):
            pltpu.sync_copy(cntv.at[pl.ds(r * 128, 128)],
                            cntS.at[iotav.at[r]], add=True)

        plsc.subcore_barrier()
        W = K * K // NS
        pltpu.sync_copy(bS.at[pl.ds(sid * W, W)],
                        b_hbm.at[cid, pl.ds(sid * W, W)])

        @pl.when(sid == 0)
        def _():
            pltpu.sync_copy(cntS, cnt_hbm.at[cid])

    return k(srcM, dstM, normM, codes1, codes2, iota2)


def _sc_recon(codes2, lsm):
    RPT = NP // NW
    CK = 64

    @_mesh(
        out_type=jax.ShapeDtypeStruct((NP, D), jnp.float32),
        scratch_types=[
            pltpu.VMEM((RPT,), jnp.int32),
            pltpu.VMEM((CK, D), jnp.float32),
            pltpu.SemaphoreType.DMA,
        ],
    )
    def k(c2_hbm, lsm_hbm, out_hbm, c2v, bufv, sem):
        cid = lax.axis_index("c")
        sid = lax.axis_index("s")
        wid = cid * NS + sid
        pltpu.sync_copy(c2_hbm.at[pl.ds(wid * RPT, RPT)], c2v)
        for kk in range(RPT // CK):
            pltpu.async_copy(lsm_hbm.at[c2v.at[pl.ds(kk * CK, CK)]],
                             bufv, sem).wait()
            pltpu.sync_copy(bufv,
                            out_hbm.at[pl.ds(wid * RPT + kk * CK, CK)])

    return k(codes2, lsm)


def _tc_t(xp, proj1):
    def body(x_ref, p_ref, o_ref):
        o_ref[...] = lax.dot_general(
            x_ref[...], p_ref[...], (((1,), (0,)), ((), ())),
            precision=HI, preferred_element_type=jnp.float32)

    return pl.pallas_call(
        body,
        out_shape=jax.ShapeDtypeStruct((NP, H), jnp.float32),
    )(xp, proj1)


def _tc_dinv(degM):
    def body(d_ref, o_ref):
        d = d_ref[...]
        o_ref[...] = jnp.where(d > 0, lax.rsqrt(jnp.maximum(d, 1e-12)), 0.0)

    return pl.pallas_call(
        body,
        out_shape=jax.ShapeDtypeStruct(degM.shape, jnp.float32),
    )(degM)


def _tc_codes(htp3):
    def body(h_ref, o_ref):
        a = h_ref[...]
        c = jnp.zeros((NP // 128, 128), jnp.int32)
        for j in range(H):
            bit = ((a[j] + a[H + j]) > 0).astype(jnp.int32)
            c = c + (bit << j)
        o_ref[...] = c

    return pl.pallas_call(
        body,
        out_shape=jax.ShapeDtypeStruct((NP // 128, 128), jnp.int32),
    )(htp3)


def _tc_stage1(M1h, xpad2, counts1p, W1, b1, proj2):
    NB = 10
    ngrid = NC * NB

    def body(m_ref, x_ref, cnt_ref, w1_ref, b1_ref, p2_ref,
             hc_ref, u_ref, acc_ref):
        i = pl.program_id(0)

        @pl.when(i == 0)
        def _():
            acc_ref[...] = jnp.zeros((K, D), jnp.float32)

        acc_ref[...] += lax.dot_general(
            m_ref[0], x_ref[0], (((1,), (0,)), ((), ())),
            precision=HI, preferred_element_type=jnp.float32)

        @pl.when(i == ngrid - 1)
        def _():
            cnt = cnt_ref[0] + cnt_ref[1]
            means = acc_ref[...] / jnp.clip(cnt, 1.0)[:, None]
            hc = lax.dot_general(
                means, w1_ref[...], (((1,), (1,)), ((), ())),
                precision=HI, preferred_element_type=jnp.float32)
            hc = jnp.maximum(hc + b1_ref[...][None, :], 0.0)
            hc_ref[...] = hc
            u_ref[...] = lax.dot_general(
                hc, p2_ref[...], (((1,), (0,)), ((), ())),
                precision=HI, preferred_element_type=jnp.float32)

    blk = HALF // NB
    return pl.pallas_call(
        body,
        grid=(ngrid,),
        in_specs=[
            pl.BlockSpec((1, K, blk), lambda i: (i // NB, 0, i % NB)),
            pl.BlockSpec((1, blk, D), lambda i: (i // NB, i % NB, 0)),
            pl.BlockSpec((NC, K), lambda i: (0, 0)),
            pl.BlockSpec((D, D), lambda i: (0, 0)),
            pl.BlockSpec((D,), lambda i: (0,)),
            pl.BlockSpec((D, H), lambda i: (0, 0)),
        ],
        out_specs=[
            pl.BlockSpec((K, D), lambda i: (0, 0)),
            pl.BlockSpec((K, H), lambda i: (0, 0)),
        ],
        out_shape=[
            jax.ShapeDtypeStruct((K, D), jnp.float32),
            jax.ShapeDtypeStruct((K, H), jnp.float32),
        ],
        scratch_shapes=[pltpu.VMEM((K, D), jnp.float32)],
    )(M1h, xpad2, counts1p, W1, b1, proj2)


def _tc_final(Bp, counts2p, hc, W2, b2):
    def body(b_ref, cnt_ref, hc_ref, w2_ref, b2_ref, o_ref):
        Bs = b_ref[0] + b_ref[1]
        sums2 = lax.dot_general(
            Bs, hc_ref[...], (((1,), (0,)), ((), ())),
            precision=HI, preferred_element_type=jnp.float32)
        cnt = cnt_ref[0] + cnt_ref[1]
        m2 = sums2 / jnp.clip(cnt, 1.0)[:, None]
        h2 = lax.dot_general(
            m2, w2_ref[...], (((1,), (1,)), ((), ())),
            precision=HI, preferred_element_type=jnp.float32)
        h2 = h2 + b2_ref[...][None, :]
        mx = jnp.max(h2, axis=1, keepdims=True)
        lse = jnp.log(jnp.sum(jnp.exp(h2 - mx), axis=1, keepdims=True)) + mx
        o_ref[...] = h2 - lse

    return pl.pallas_call(
        body,
        out_shape=jax.ShapeDtypeStruct((K, D), jnp.float32),
    )(Bp, counts2p, hc, W2, b2)


def kernel(x, edge_index, edge_weight, W1, b1, W2, b2, proj1, proj2):
    src = edge_index[0]
    dst = edge_index[1]
    pad = EP - E
    srcM = jnp.concatenate([src, jnp.zeros((pad,), jnp.int32)]).reshape(ROWS, 128)
    dstM = jnp.concatenate([dst, jnp.zeros((pad,), jnp.int32)]).reshape(ROWS, 128)
    wM = jnp.concatenate(
        [edge_weight, jnp.zeros((pad,), jnp.float32)]).reshape(ROWS, 128)
    xp = jnp.concatenate([x, jnp.zeros((NP - N, D), jnp.float32)])
    iota2 = jnp.arange(K, dtype=jnp.int32).reshape(2, 128)

    degp = _sc_deg(dstM, wM)
    dinv = _tc_dinv(degp.reshape(NP // 128, 128)).reshape(NP)
    t = _tc_t(xp, proj1)
    tflat = t.reshape(NP * H)

    normM, htp = _sc_norm_ht(srcM, dstM, wM, dinv, tflat)
    codes1 = _tc_codes(htp.reshape(NC * H, NP // 128, 128)).reshape(NP)

    m1flat, counts1p = _sc_m1(srcM, dstM, normM, codes1, iota2)
    hc, u = _tc_stage1(m1flat.reshape(NC, K, HALF), xp.reshape(NC, HALF, D),
                       counts1p, W1, b1, proj2)

    gtp = _sc_gt(srcM, dstM, normM, codes1, u.reshape(K * H))
    codes2 = _tc_codes(gtp.reshape(NC * H, NP // 128, 128)).reshape(NP)

    bflat, counts2p = _sc_b(srcM, dstM, normM, codes1, codes2, iota2)
    lsm = _tc_final(bflat.reshape(NC, K, K), counts2p, hc, W2, b2)

    outp = _sc_recon(codes2, lsm)
    return outp[:N]


# speedup vs baseline: 14.1181x; 1.0895x over previous
import dataclasses
import functools

import jax
import jax.numpy as jnp
from jax import lax
from jax.experimental import pallas as pl
from jax.experimental.pallas import tpu as pltpu
from jax.experimental.pallas import tpu_sc as plsc

N = 10000
E = 320000
D = 128
H = 8
K = 256

NC, NS, L = 2, 16, 16
NW = NC * NS
NP = 10240
HALF = NP // NC
ROWS = 2560
EP = ROWS * 128
CH = 16
NCHUNK = ROWS // CH
HI = jax.lax.Precision.HIGHEST

_cp = pltpu.CompilerParams()
if "needs_layout_passes" in pltpu.CompilerParams.__dataclass_fields__:
    _cp = dataclasses.replace(_cp, needs_layout_passes=False)

_mesh = functools.partial(
    pl.kernel,
    mesh=plsc.VectorSubcoreMesh(core_axis_name="c", subcore_axis_name="s"),
    compiler_params=_cp,
)


def _zero_shared(shared_ref, zbuf, sid, nwords_per_tile):
    nz = zbuf.shape[0]

    @pl.loop(0, nz // L)
    def _(i):
        zbuf[pl.ds(i * L, L)] = jnp.zeros((L,), jnp.float32)

    @pl.loop(0, nwords_per_tile // nz)
    def _(i):
        pltpu.sync_copy(
            zbuf, shared_ref.at[pl.ds(sid * nwords_per_tile + i * nz, nz)])


def _sc_deg(dstM, wM):
    @_mesh(
        out_type=jax.ShapeDtypeStruct((NP,), jnp.float32),
        scratch_types=[
            pltpu.VMEM((CH, 128), jnp.int32),
            pltpu.VMEM((CH, 128), jnp.float32),
            pltpu.VMEM((640,), jnp.float32),
            pltpu.VMEM_SHARED((NP,), jnp.float32),
            pltpu.SemaphoreType.DMA,
            pltpu.SemaphoreType.DMA,
        ],
    )
    def k(dst_hbm, w_hbm, deg_hbm, dstv, wv, zbuf, degS, sem, sem2):
        cid = lax.axis_index("c")
        sid = lax.axis_index("s")
        _zero_shared(degS, zbuf, sid, NP // NS)
        plsc.subcore_barrier()

        @pl.loop(0, NCHUNK // NS)
        def _(i):
            row0 = (sid * (NCHUNK // NS) + i) * CH
            pltpu.async_copy(dst_hbm.at[pl.ds(row0, CH)], dstv, sem).wait()
            pltpu.async_copy(w_hbm.at[pl.ds(row0, CH)], wv, sem).wait()
            ds_ = [pltpu.async_copy(wv.at[r], degS.at[dstv.at[r]], sem2,
                                    add=True) for r in range(CH)]
            for d in ds_:
                d.wait()

        plsc.subcore_barrier()
        pltpu.sync_copy(degS.at[pl.ds(cid * HALF, HALF)],
                        deg_hbm.at[pl.ds(cid * HALF, HALF)])

    return k(dstM, wM)


def _sc_norm_ht(srcM, dstM, wM, dinv, tflat):
    @_mesh(
        out_type=(
            jax.ShapeDtypeStruct((ROWS, 128), jnp.float32),
            jax.ShapeDtypeStruct((NC, H * NP), jnp.float32),
        ),
        scratch_types=[
            pltpu.VMEM((CH, 128), jnp.int32),
            pltpu.VMEM((CH, 128), jnp.int32),
            pltpu.VMEM((CH, 128), jnp.float32),
            pltpu.VMEM((CH, 128), jnp.float32),
            pltpu.VMEM((NP,), jnp.float32),
            pltpu.VMEM((NP * H,), jnp.float32),
            pltpu.VMEM((H, CH, 128), jnp.float32),
            pltpu.VMEM((640,), jnp.float32),
        ] + [pltpu.VMEM_SHARED((NP,), jnp.float32) for _ in range(H)]
        + [pltpu.SemaphoreType.DMA],
    )
    def k(src_hbm, dst_hbm, w_hbm, dinv_hbm, t_hbm, norm_hbm, ht_hbm,
          srcv, dstv, wv, normv, dinvv, tv, stage, zbuf, *rest):
        hts, sem = rest[:H], rest[H]
        cid = lax.axis_index("c")
        sid = lax.axis_index("s")
        for j in range(H):
            _zero_shared(hts[j], zbuf, sid, NP // NS)
        pltpu.async_copy(dinv_hbm, dinvv, sem).wait()
        pltpu.async_copy(t_hbm, tv, sem).wait()
        plsc.subcore_barrier()

        nch = NCHUNK // NW

        @pl.loop(0, nch)
        def _(i):
            row0 = (cid * (NCHUNK // NC) + sid * nch + i) * CH
            pltpu.async_copy(src_hbm.at[pl.ds(row0, CH)], srcv, sem).wait()
            pltpu.async_copy(dst_hbm.at[pl.ds(row0, CH)], dstv, sem).wait()
            pltpu.async_copy(w_hbm.at[pl.ds(row0, CH)], wv, sem).wait()

            @plsc.parallel_loop(0, CH * 8, unroll=4)
            def _(q):
                r = q >> 3
                c0 = (q & 7) * L
                s16 = srcv[r, pl.ds(c0, L)]
                d16 = dstv[r, pl.ds(c0, L)]
                w16 = wv[r, pl.ds(c0, L)]
                nrm = (plsc.load_gather(dinvv, [s16]) * w16
                       * plsc.load_gather(dinvv, [d16]))
                normv[r, pl.ds(c0, L)] = nrm
                s8 = s16 * 8
                for j in range(H):
                    tvj = plsc.load_gather(tv, [s8 + j])
                    stage[j, r, pl.ds(c0, L)] = tvj * nrm

            pltpu.async_copy(normv, norm_hbm.at[pl.ds(row0, CH)], sem).wait()

            @pl.loop(0, CH)
            def _(r):
                ds_ = [pltpu.async_copy(stage.at[j, r],
                                        hts[j].at[dstv.at[r]],
                                        sem, add=True) for j in range(H)]
                for d in ds_:
                    d.wait()

        plsc.subcore_barrier()
        for j in range(H):
            pltpu.sync_copy(
                hts[j].at[pl.ds(sid * (NP // NS), NP // NS)],
                ht_hbm.at[cid, pl.ds(j * NP + sid * (NP // NS), NP // NS)])

    return k(srcM, dstM, wM, dinv, tflat)


def _sc_m1(srcM, dstM, normM, codes1, iota2):
    @_mesh(
        out_type=(
            jax.ShapeDtypeStruct((NC, K * HALF), jnp.float32),
            jax.ShapeDtypeStruct((NC, K), jnp.float32),
        ),
        scratch_types=[
            pltpu.VMEM((CH, 128), jnp.int32),
            pltpu.VMEM((CH, 128), jnp.int32),
            pltpu.VMEM((CH, 128), jnp.float32),
            pltpu.VMEM((CH, 128), jnp.int32),
            pltpu.VMEM((CH, 128), jnp.float32),
            pltpu.VMEM((NP,), jnp.int32),
            pltpu.VMEM((2, 128), jnp.int32),
            pltpu.VMEM((K,), jnp.float32),
            pltpu.VMEM((8192,), jnp.float32),
            pltpu.VMEM_SHARED((K * HALF,), jnp.float32),
            pltpu.VMEM_SHARED((K,), jnp.float32),
            pltpu.SemaphoreType.DMA,
        ],
    )
    def k(src_hbm, dst_hbm, norm_hbm, c1_hbm, iota_hbm, m1_hbm, cnt_hbm,
          srcv, dstv, normv, flatv, valv, c1v, iotav, cntv, zbuf,
          m1S, cntS, sem):
        cid = lax.axis_index("c")
        sid = lax.axis_index("s")
        _zero_shared(m1S, zbuf, sid, K * HALF // NS)

        @pl.when(sid == 0)
        def _():
            @pl.loop(0, K // L)
            def _(i):
                zbuf[pl.ds(i * L, L)] = jnp.zeros((L,), jnp.float32)
            pltpu.sync_copy(zbuf.at[pl.ds(0, K)], cntS)

        pltpu.async_copy(c1_hbm, c1v, sem).wait()
        pltpu.async_copy(iota_hbm, iotav, sem).wait()
        plsc.subcore_barrier()
        base = cid * HALF

        @pl.loop(0, K // L)
        def _(i):
            cntv[pl.ds(i * L, L)] = jnp.zeros((L,), jnp.float32)

        wid = cid * NS + sid

        @pl.loop(0, (NP // NW) // L)
        def _(i):
            n0 = wid * (NP // NW) + i * L
            c16 = c1v[pl.ds(n0, L)]
            valid = (lax.iota(jnp.int32, L) + n0) < N
            plsc.addupdate_scatter(
                cntv, [c16], jnp.where(valid, 1.0, 0.0))

        @pl.loop(0, NCHUNK // NS)
        def _(i):
            row0 = (sid * (NCHUNK // NS) + i) * CH
            pltpu.async_copy(src_hbm.at[pl.ds(row0, CH)], srcv, sem).wait()
            pltpu.async_copy(dst_hbm.at[pl.ds(row0, CH)], dstv, sem).wait()
            pltpu.async_copy(norm_hbm.at[pl.ds(row0, CH)], normv, sem).wait()

            @plsc.parallel_loop(0, CH * 8, unroll=4)
            def _(q):
                r = q >> 3
                c0 = (q & 7) * L
                s16 = srcv[r, pl.ds(c0, L)]
                d16 = dstv[r, pl.ds(c0, L)]
                n16 = normv[r, pl.ds(c0, L)]
                c16 = plsc.load_gather(c1v, [d16])
                col = s16 - base
                owned = (col >= 0) & (col < HALF)
                col = jnp.clip(col, 0, HALF - 1)
                flatv[r, pl.ds(c0, L)] = c16 * HALF + col
                valv[r, pl.ds(c0, L)] = jnp.where(owned, n16, 0.0)

            ds_ = [pltpu.async_copy(valv.at[r], m1S.at[flatv.at[r]], sem,
                                    add=True) for r in range(CH)]
            for d in ds_:
                d.wait()

        for r in range(2):
            pltpu.sync_copy(cntv.at[pl.ds(r * 128, 128)],
                            cntS.at[iotav.at[r]], add=True)

        plsc.subcore_barrier()
        W = K * HALF // NS
        pltpu.sync_copy(m1S.at[pl.ds(sid * W, W)],
                        m1_hbm.at[cid, pl.ds(sid * W, W)])

        @pl.when(sid == 0)
        def _():
            pltpu.sync_copy(cntS, cnt_hbm.at[cid])

    return k(srcM, dstM, normM, codes1, iota2)


def _sc_gt(srcM, dstM, normM, codes1, uflat):
    @_mesh(
        out_type=jax.ShapeDtypeStruct((NC, H * NP), jnp.float32),
        scratch_types=[
            pltpu.VMEM((CH, 128), jnp.int32),
            pltpu.VMEM((CH, 128), jnp.int32),
            pltpu.VMEM((CH, 128), jnp.float32),
            pltpu.VMEM((NP,), jnp.int32),
            pltpu.VMEM((K * H,), jnp.float32),
            pltpu.VMEM((H, CH, 128), jnp.float32),
            pltpu.VMEM((640,), jnp.float32),
        ] + [pltpu.VMEM_SHARED((NP,), jnp.float32) for _ in range(H)]
        + [pltpu.SemaphoreType.DMA],
    )
    def k(src_hbm, dst_hbm, norm_hbm, c1_hbm, u_hbm, gt_hbm,
          srcv, dstv, normv, c1v, uv, stage, zbuf, *rest):
        gts, sem = rest[:H], rest[H]
        cid = lax.axis_index("c")
        sid = lax.axis_index("s")
        for j in range(H):
            _zero_shared(gts[j], zbuf, sid, NP // NS)
        pltpu.async_copy(c1_hbm, c1v, sem).wait()
        pltpu.async_copy(u_hbm, uv, sem).wait()
        plsc.subcore_barrier()

        nch = NCHUNK // NW

        @pl.loop(0, nch)
        def _(i):
            row0 = (cid * (NCHUNK // NC) + sid * nch + i) * CH
            pltpu.async_copy(src_hbm.at[pl.ds(row0, CH)], srcv, sem).wait()
            pltpu.async_copy(dst_hbm.at[pl.ds(row0, CH)], dstv, sem).wait()
            pltpu.async_copy(norm_hbm.at[pl.ds(row0, CH)], normv, sem).wait()

            @plsc.parallel_loop(0, CH * 8, unroll=4)
            def _(q):
                r = q >> 3
                c0 = (q & 7) * L
                s16 = srcv[r, pl.ds(c0, L)]
                n16 = normv[r, pl.ds(c0, L)]
                k16 = plsc.load_gather(c1v, [s16]) * 8
                for j in range(H):
                    uvj = plsc.load_gather(uv, [k16 + j])
                    stage[j, r, pl.ds(c0, L)] = uvj * n16

            @pl.loop(0, CH)
            def _(r):
                ds_ = [pltpu.async_copy(stage.at[j, r],
                                        gts[j].at[dstv.at[r]],
                                        sem, add=True) for j in range(H)]
                for d in ds_:
                    d.wait()

        plsc.subcore_barrier()
        for j in range(H):
            pltpu.sync_copy(
                gts[j].at[pl.ds(sid * (NP // NS), NP // NS)],
                gt_hbm.at[cid, pl.ds(j * NP + sid * (NP // NS), NP // NS)])

    return k(srcM, dstM, normM, codes1, uflat)


def _sc_b(srcM, dstM, normM, codes1, codes2, iota2):
    @_mesh(
        out_type=(
            jax.ShapeDtypeStruct((NC, K * K), jnp.float32),
            jax.ShapeDtypeStruct((NC, K), jnp.float32),
        ),
        scratch_types=[
            pltpu.VMEM((CH, 128), jnp.int32),
            pltpu.VMEM((CH, 128), jnp.int32),
            pltpu.VMEM((CH, 128), jnp.float32),
            pltpu.VMEM((CH, 128), jnp.int32),
            pltpu.VMEM((NP,), jnp.int32),
            pltpu.VMEM((NP,), jnp.int32),
            pltpu.VMEM((2, 128), jnp.int32),
            pltpu.VMEM((K,), jnp.float32),
            pltpu.VMEM((4096,), jnp.float32),
            pltpu.VMEM_SHARED((K * K,), jnp.float32),
            pltpu.VMEM_SHARED((K,), jnp.float32),
            pltpu.SemaphoreType.DMA,
        ],
    )
    def k(src_hbm, dst_hbm, norm_hbm, c1_hbm, c2_hbm, iota_hbm,
          b_hbm, cnt_hbm,
          srcv, dstv, normv, flatv, c1v, c2v, iotav, cntv, zbuf,
          bS, cntS, sem):
        cid = lax.axis_index("c")
        sid = lax.axis_index("s")
        _zero_shared(bS, zbuf, sid, K * K // NS)

        @pl.when(sid == 0)
        def _():
            pltpu.sync_copy(zbuf.at[pl.ds(0, K)], cntS)

        pltpu.async_copy(c1_hbm, c1v, sem).wait()
        pltpu.async_copy(c2_hbm, c2v, sem).wait()
        pltpu.async_copy(iota_hbm, iotav, sem).wait()
        plsc.subcore_barrier()

        @pl.loop(0, K // L)
        def _(i):
            cntv[pl.ds(i * L, L)] = jnp.zeros((L,), jnp.float32)

        wid = cid * NS + sid

        @pl.loop(0, (NP // NW) // L)
        def _(i):
            n0 = wid * (NP // NW) + i * L
            c16 = c2v[pl.ds(n0, L)]
            valid = (lax.iota(jnp.int32, L) + n0) < N
            plsc.addupdate_scatter(
                cntv, [c16], jnp.where(valid, 1.0, 0.0))

        nch = NCHUNK // NW

        @pl.loop(0, nch)
        def _(i):
            row0 = (cid * (NCHUNK // NC) + sid * nch + i) * CH
            pltpu.async_copy(src_hbm.at[pl.ds(row0, CH)], srcv, sem).wait()
            pltpu.async_copy(dst_hbm.at[pl.ds(row0, CH)], dstv, sem).wait()
            pltpu.async_copy(norm_hbm.at[pl.ds(row0, CH)], normv, sem).wait()

            @plsc.parallel_loop(0, CH * 8, unroll=4)
            def _(q):
                r = q >> 3
                c0 = (q & 7) * L
                s16 = srcv[r, pl.ds(c0, L)]
                d16 = dstv[r, pl.ds(c0, L)]
                k1 = plsc.load_gather(c1v, [s16])
                k2 = plsc.load_gather(c2v, [d16])
                flatv[r, pl.ds(c0, L)] = k2 * K + k1

            ds_ = [pltpu.async_copy(normv.at[r], bS.at[flatv.at[r]], sem,
                                    add=True) for r in range(CH)]
            for d in ds_:
                d.wait()

        for r in range(2):
            pltpu.sync_copy(cntv.at[pl.ds(r * 128, 128)],
                            cntS.at[iotav.at[r]], add=True)

        plsc.subcore_barrier()
        W = K * K // NS
        pltpu.sync_copy(bS.at[pl.ds(sid * W, W)],
                        b_hbm.at[cid, pl.ds(sid * W, W)])

        @pl.when(sid == 0)
        def _():
            pltpu.sync_copy(cntS, cnt_hbm.at[cid])

    return k(srcM, dstM, normM, codes1, codes2, iota2)


def _sc_recon(codes2, lsm):
    RPT = NP // NW
    CK = 64

    @_mesh(
        out_type=jax.ShapeDtypeStruct((NP, D), jnp.float32),
        scratch_types=[
            pltpu.VMEM((RPT,), jnp.int32),
            pltpu.VMEM((CK, D), jnp.float32),
            pltpu.SemaphoreType.DMA,
        ],
    )
    def k(c2_hbm, lsm_hbm, out_hbm, c2v, bufv, sem):
        cid = lax.axis_index("c")
        sid = lax.axis_index("s")
        wid = cid * NS + sid
        pltpu.sync_copy(c2_hbm.at[pl.ds(wid * RPT, RPT)], c2v)
        for kk in range(RPT // CK):
            pltpu.async_copy(lsm_hbm.at[c2v.at[pl.ds(kk * CK, CK)]],
                             bufv, sem).wait()
            pltpu.sync_copy(bufv,
                            out_hbm.at[pl.ds(wid * RPT + kk * CK, CK)])

    return k(codes2, lsm)


def _tc_t(xp, proj1):
    def body(x_ref, p_ref, o_ref):
        o_ref[...] = lax.dot_general(
            x_ref[...], p_ref[...], (((1,), (0,)), ((), ())),
            precision=HI, preferred_element_type=jnp.float32)

    return pl.pallas_call(
        body,
        out_shape=jax.ShapeDtypeStruct((NP, H), jnp.float32),
    )(xp, proj1)


def _tc_dinv(degM):
    def body(d_ref, o_ref):
        d = d_ref[...]
        o_ref[...] = jnp.where(d > 0, lax.rsqrt(jnp.maximum(d, 1e-12)), 0.0)

    return pl.pallas_call(
        body,
        out_shape=jax.ShapeDtypeStruct(degM.shape, jnp.float32),
    )(degM)


def _tc_codes(htp3):
    def body(h_ref, o_ref):
        a = h_ref[...]
        c = jnp.zeros((NP // 128, 128), jnp.int32)
        for j in range(H):
            bit = ((a[j] + a[H + j]) > 0).astype(jnp.int32)
            c = c + (bit << j)
        o_ref[...] = c

    return pl.pallas_call(
        body,
        out_shape=jax.ShapeDtypeStruct((NP // 128, 128), jnp.int32),
    )(htp3)


def _tc_stage1(M1h, xpad2, counts1p, W1, b1, proj2):
    NB = 10
    ngrid = NC * NB

    def body(m_ref, x_ref, cnt_ref, w1_ref, b1_ref, p2_ref,
             hc_ref, u_ref, acc_ref):
        i = pl.program_id(0)

        @pl.when(i == 0)
        def _():
            acc_ref[...] = jnp.zeros((K, D), jnp.float32)

        acc_ref[...] += lax.dot_general(
            m_ref[0], x_ref[0], (((1,), (0,)), ((), ())),
            precision=HI, preferred_element_type=jnp.float32)

        @pl.when(i == ngrid - 1)
        def _():
            cnt = cnt_ref[0] + cnt_ref[1]
            means = acc_ref[...] / jnp.clip(cnt, 1.0)[:, None]
            hc = lax.dot_general(
                means, w1_ref[...], (((1,), (1,)), ((), ())),
                precision=HI, preferred_element_type=jnp.float32)
            hc = jnp.maximum(hc + b1_ref[...][None, :], 0.0)
            hc_ref[...] = hc
            u_ref[...] = lax.dot_general(
                hc, p2_ref[...], (((1,), (0,)), ((), ())),
                precision=HI, preferred_element_type=jnp.float32)

    blk = HALF // NB
    return pl.pallas_call(
        body,
        grid=(ngrid,),
        in_specs=[
            pl.BlockSpec((1, K, blk), lambda i: (i // NB, 0, i % NB)),
            pl.BlockSpec((1, blk, D), lambda i: (i // NB, i % NB, 0)),
            pl.BlockSpec((NC, K), lambda i: (0, 0)),
            pl.BlockSpec((D, D), lambda i: (0, 0)),
            pl.BlockSpec((D,), lambda i: (0,)),
            pl.BlockSpec((D, H), lambda i: (0, 0)),
        ],
        out_specs=[
            pl.BlockSpec((K, D), lambda i: (0, 0)),
            pl.BlockSpec((K, H), lambda i: (0, 0)),
        ],
        out_shape=[
            jax.ShapeDtypeStruct((K, D), jnp.float32),
            jax.ShapeDtypeStruct((K, H), jnp.float32),
        ],
        scratch_shapes=[pltpu.VMEM((K, D), jnp.float32)],
    )(M1h, xpad2, counts1p, W1, b1, proj2)


def _tc_final(Bp, counts2p, hc, W2, b2):
    def body(b_ref, cnt_ref, hc_ref, w2_ref, b2_ref, o_ref):
        Bs = b_ref[0] + b_ref[1]
        sums2 = lax.dot_general(
            Bs, hc_ref[...], (((1,), (0,)), ((), ())),
            precision=HI, preferred_element_type=jnp.float32)
        cnt = cnt_ref[0] + cnt_ref[1]
        m2 = sums2 / jnp.clip(cnt, 1.0)[:, None]
        h2 = lax.dot_general(
            m2, w2_ref[...], (((1,), (1,)), ((), ())),
            precision=HI, preferred_element_type=jnp.float32)
        h2 = h2 + b2_ref[...][None, :]
        mx = jnp.max(h2, axis=1, keepdims=True)
        lse = jnp.log(jnp.sum(jnp.exp(h2 - mx), axis=1, keepdims=True)) + mx
        o_ref[...] = h2 - lse

    return pl.pallas_call(
        body,
        out_shape=jax.ShapeDtypeStruct((K, D), jnp.float32),
    )(Bp, counts2p, hc, W2, b2)


def kernel(x, edge_index, edge_weight, W1, b1, W2, b2, proj1, proj2):
    src = edge_index[0]
    dst = edge_index[1]
    pad = EP - E
    srcM = jnp.concatenate([src, jnp.zeros((pad,), jnp.int32)]).reshape(ROWS, 128)
    dstM = jnp.concatenate([dst, jnp.zeros((pad,), jnp.int32)]).reshape(ROWS, 128)
    wM = jnp.concatenate(
        [edge_weight, jnp.zeros((pad,), jnp.float32)]).reshape(ROWS, 128)
    xp = jnp.concatenate([x, jnp.zeros((NP - N, D), jnp.float32)])
    iota2 = jnp.arange(K, dtype=jnp.int32).reshape(2, 128)

    degp = _sc_deg(dstM, wM)
    dinv = _tc_dinv(degp.reshape(NP // 128, 128)).reshape(NP)
    t = _tc_t(xp, proj1)
    tflat = t.reshape(NP * H)

    normM, htp = _sc_norm_ht(srcM, dstM, wM, dinv, tflat)
    codes1 = _tc_codes(htp.reshape(NC * H, NP // 128, 128)).reshape(NP)

    m1flat, counts1p = _sc_m1(srcM, dstM, normM, codes1, iota2)
    hc, u = _tc_stage1(m1flat.reshape(NC, K, HALF), xp.reshape(NC, HALF, D),
                       counts1p, W1, b1, proj2)

    gtp = _sc_gt(srcM, dstM, normM, codes1, u.reshape(K * H))
    codes2 = _tc_codes(gtp.reshape(NC * H, NP // 128, 128)).reshape(NP)

    bflat, counts2p = _sc_b(srcM, dstM, normM, codes1, codes2, iota2)
    lsm = _tc_final(bflat.reshape(NC, K, K), counts2p, hc, W2, b2)

    outp = _sc_recon(codes2, lsm)
    return outp[:N]
